# baseline (speedup 1.0000x reference)
import jax
import jax.numpy as jnp
from jax import lax
from jax.experimental import pallas as pl
from jax.experimental.pallas import tpu as pltpu
from jax.experimental.pallas import tpu_sc as plsc

_SPLIT = 50000


def _tc_copy_kernel(in_ref, out_ref):
    out_ref[...] = in_ref[...]


def _tc_copy(x):
    n, d = x.shape
    blk = 10000
    return pl.pallas_call(
        _tc_copy_kernel,
        grid=(pl.cdiv(n, blk),),
        in_specs=[pl.BlockSpec((blk, d), lambda i: (i, 0))],
        out_specs=pl.BlockSpec((blk, d), lambda i: (i, 0)),
        out_shape=jax.ShapeDtypeStruct(x.shape, x.dtype),
    )(x)


def _sc_copy(flat):
    tot = flat.shape[0]
    mesh = plsc.VectorSubcoreMesh(core_axis_name="c", subcore_axis_name="s")
    nc, nw = mesh.num_cores, mesh.size
    chunk = -(-tot // nw)
    chunk = -(-chunk // 16) * 16
    offmax = tot - chunk

    def body(in_hbm, out_hbm, buf):
        wid = lax.axis_index("s") * nc + lax.axis_index("c")
        off = jnp.minimum(wid * chunk, offmax)
        pltpu.sync_copy(in_hbm.at[pl.ds(off, chunk)], buf)
        pltpu.sync_copy(buf, out_hbm.at[pl.ds(off, chunk)])

    return pl.kernel(
        body,
        out_type=jax.ShapeDtypeStruct((tot,), flat.dtype),
        mesh=mesh,
        scratch_types=[pltpu.VMEM((chunk,), flat.dtype)],
    )(flat)


def kernel(xyz, reconstruction, neighborsMatrix, numNeighbors, weightMatrix, arapWeight):
    n, d = reconstruction.shape
    h1 = reconstruction[:_SPLIT]
    h2 = reconstruction[_SPLIT:].reshape(-1)
    o1 = _tc_copy(h1)
    o2 = _sc_copy(h2)
    return jnp.concatenate([o1, o2.reshape(n - _SPLIT, d)], axis=0)


# speedup vs baseline: 1.6333x; 1.6333x over previous
import jax
import jax.numpy as jnp
from jax.experimental import pallas as pl
from jax.experimental.pallas import tpu as pltpu

_CH = 8
_R = 12500


def _copy_kernel(in_ref, out_ref, *scratch):
    bufs = scratch[:_CH]
    sin = scratch[_CH:2 * _CH]
    sout = scratch[2 * _CH:3 * _CH]
    ins = [pltpu.make_async_copy(in_ref.at[pl.ds(i * _R, _R), :], bufs[i], sin[i])
           for i in range(_CH)]
    outs = [pltpu.make_async_copy(bufs[i], out_ref.at[pl.ds(i * _R, _R), :], sout[i])
            for i in range(_CH)]
    for c in ins:
        c.start()
    for i in range(_CH):
        ins[i].wait()
        outs[i].start()
    for c in outs:
        c.wait()


def kernel(xyz, reconstruction, neighborsMatrix, numNeighbors, weightMatrix, arapWeight):
    n, d = reconstruction.shape
    return pl.pallas_call(
        _copy_kernel,
        out_shape=jax.ShapeDtypeStruct(reconstruction.shape, reconstruction.dtype),
        in_specs=[pl.BlockSpec(memory_space=pltpu.MemorySpace.HBM)],
        out_specs=pl.BlockSpec(memory_space=pltpu.MemorySpace.HBM),
        scratch_shapes=(
            [pltpu.VMEM((_R, d), reconstruction.dtype)] * _CH
            + [pltpu.SemaphoreType.DMA] * (2 * _CH)
        ),
    )(reconstruction)


# speedup vs baseline: 1.6472x; 1.0085x over previous
import jax
import jax.numpy as jnp
from jax.experimental import pallas as pl


def _copy_kernel(in_ref, out_ref):
    out_ref[...] = in_ref[...]


def kernel(xyz, reconstruction, neighborsMatrix, numNeighbors, weightMatrix, arapWeight):
    n, d = reconstruction.shape
    blk = 25000
    return pl.pallas_call(
        _copy_kernel,
        grid=(pl.cdiv(n, blk),),
        in_specs=[pl.BlockSpec((blk, d), lambda i: (i, 0))],
        out_specs=pl.BlockSpec((blk, d), lambda i: (i, 0)),
        out_shape=jax.ShapeDtypeStruct(reconstruction.shape, reconstruction.dtype),
    )(reconstruction)
